# baseline (speedup 1.0000x reference)
import functools

import jax
import jax.numpy as jnp
from jax import lax
from jax.experimental import pallas as pl
from jax.experimental.pallas import tpu as pltpu
from jax.experimental.pallas import tpu_sc as plsc

N = 10000
P = 10240
E = 320000
D = 128
G = 64
NW = 32
CHUNK = 128
NCH0 = 160
NCH1 = 0
TOTCH = 16 * (NCH0 + NCH1)
RPT = P // 16


@functools.lru_cache(maxsize=None)
def _sc_kernels():
    mesh = plsc.VectorSubcoreMesh(core_axis_name="c", subcore_axis_name="s")

    @functools.partial(
        pl.kernel,
        out_type=jax.ShapeDtypeStruct((2, P, D), jnp.float32),
        mesh=mesh,
        scratch_types=[
            pltpu.VMEM((4, CHUNK), jnp.int32),
            pltpu.VMEM((4, CHUNK), jnp.int32),
            pltpu.VMEM((CHUNK, D), jnp.float32),
            pltpu.VMEM((CHUNK, D), jnp.float32),
            pltpu.VMEM_SHARED((P, D), jnp.float32),
            pltpu.SemaphoreType.DMA,
            pltpu.SemaphoreType.DMA,
            pltpu.SemaphoreType.DMA,
            pltpu.SemaphoreType.DMA,
        ],
    )
    def _agg_kernel(h_hbm, src_hbm, dst_hbm, zeros_hbm, out_hbm,
                    sring, dring, rows_a, rows_b, acc,
                    gsem_a, gsem_b, isem_s, isem_d):
        cid = lax.axis_index("c")
        sid = lax.axis_index("s")
        base = jnp.where(cid == 0, sid * NCH0, 16 * NCH0 + sid * NCH1)
        npair = jnp.where(cid == 0, NCH0 // 2, NCH1 // 2)
        pltpu.sync_copy(zeros_hbm, acc.at[pl.ds(sid * RPT, RPT)])
        pltpu.sync_copy(src_hbm.at[pl.ds(base, 2)], sring.at[pl.ds(0, 2)])
        pltpu.sync_copy(dst_hbm.at[pl.ds(base, 2)], dring.at[pl.ds(0, 2)])
        plsc.subcore_barrier()

        @pl.when(npair > 0)
        def _():
            pltpu.async_copy(h_hbm.at[sring.at[0]], rows_a, gsem_a)

        @pl.loop(0, npair)
        def _(k):
            j = base + k * 2
            p = (k % 2) * 2
            q = 2 - p
            more = k + 1 < npair

            @pl.when(more)
            def _():
                pltpu.async_copy(src_hbm.at[pl.ds(j + 2, 2)],
                                 sring.at[pl.ds(q, 2)], isem_s)
                pltpu.async_copy(dst_hbm.at[pl.ds(j + 2, 2)],
                                 dring.at[pl.ds(q, 2)], isem_d)

            pltpu.async_copy(h_hbm.at[sring.at[p + 1]], rows_b, gsem_b)
            pltpu.make_async_copy(h_hbm.at[sring.at[p]], rows_a,
                                  gsem_a).wait()
            pltpu.sync_copy(rows_a, acc.at[dring.at[p]], add=True)

            @pl.when(more)
            def _():
                pltpu.make_async_copy(src_hbm.at[pl.ds(j + 2, 2)],
                                      sring.at[pl.ds(q, 2)], isem_s).wait()
                pltpu.async_copy(h_hbm.at[sring.at[q]], rows_a, gsem_a)

            pltpu.make_async_copy(h_hbm.at[sring.at[p + 1]], rows_b,
                                  gsem_b).wait()
            pltpu.sync_copy(rows_b, acc.at[dring.at[p + 1]], add=True)

            @pl.when(more)
            def _():
                pltpu.make_async_copy(dst_hbm.at[pl.ds(j + 2, 2)],
                                      dring.at[pl.ds(q, 2)], isem_d).wait()

        plsc.subcore_barrier()
        pltpu.sync_copy(acc.at[pl.ds(sid * RPT, RPT)],
                        out_hbm.at[cid, pl.ds(sid * RPT, RPT)])

    return _agg_kernel


def _tc1_body(x_ref, w_ref, degs_ref, h_ref, dinv_ref):
    deg = degs_ref[0, :, 0:1] + degs_ref[1, :, 0:1] + 1.0
    rows = lax.broadcasted_iota(jnp.int32, (P, 1), 0)
    dinv = jnp.where(rows < N, lax.rsqrt(deg), 0.0)
    h = jnp.dot(x_ref[...], w_ref[...], preferred_element_type=jnp.float32)
    h_ref[...] = h * dinv
    dinv_ref[...] = dinv


def _tc2_body(agg_ref, hp_ref, dinv_ref, b_ref, w_ref, out_ref):
    dinv = dinv_ref[...]
    s = (agg_ref[0] + agg_ref[1] + hp_ref[...]) * dinv + b_ref[...]
    h = jnp.maximum(s, 0.0)
    out_ref[...] = jnp.dot(h, w_ref[...],
                           preferred_element_type=jnp.float32) * dinv


def _tc3_body(agg_ref, hp_ref, dinv_ref, b_ref, batch_ref,
              wp1_ref, bp1_ref, wp2_ref, bp2_ref, out_ref):
    dinv = dinv_ref[...]
    h3 = jnp.maximum(
        (agg_ref[0] + agg_ref[1] + hp_ref[...]) * dinv + b_ref[...], 0.0)
    gids = lax.broadcasted_iota(jnp.int32, (G, P), 0)
    pm = (batch_ref[...] == gids).astype(jnp.float32)
    counts = jnp.sum(pm, axis=1, keepdims=True)
    summed = jnp.dot(pm, h3, preferred_element_type=jnp.float32)
    pooled = summed / jnp.maximum(counts, 1.0)
    t = jnp.maximum(
        jnp.dot(pooled, wp1_ref[...], preferred_element_type=jnp.float32)
        + bp1_ref[...], 0.0)
    out_ref[...] = jnp.dot(t, wp2_ref[...],
                           preferred_element_type=jnp.float32) + bp2_ref[...]


def _tc1(xp, W0, degs):
    return pl.pallas_call(
        _tc1_body,
        out_shape=[jax.ShapeDtypeStruct((P, D), jnp.float32),
                   jax.ShapeDtypeStruct((P, 1), jnp.float32)],
    )(xp, W0, degs)


def _tc2(agg, hp, dinv, b, Wn):
    return pl.pallas_call(
        _tc2_body,
        out_shape=jax.ShapeDtypeStruct((P, D), jnp.float32),
    )(agg, hp, dinv, b, Wn)


def _tc3(agg, hp, dinv, b, batch_p, Wp1, bp1, Wp2, bp2):
    return pl.pallas_call(
        _tc3_body,
        out_shape=jax.ShapeDtypeStruct((G, 5), jnp.float32),
    )(agg, hp, dinv, b, batch_p, Wp1, bp1, Wp2, bp2)


def kernel(x, edge_index, batch, W0, b0, W1, b1, W2, b2, Wp1, bp1, Wp2, bp2):
    ei = edge_index.astype(jnp.int32)
    npad = TOTCH * CHUNK - E
    src = jnp.concatenate([ei[0], jnp.zeros((npad,), jnp.int32)])
    dst = jnp.concatenate([ei[1], jnp.full((npad,), P - 1, jnp.int32)])
    src3 = src.reshape(TOTCH, CHUNK)
    dst3 = dst.reshape(TOTCH, CHUNK)

    ones_h = jnp.zeros((P, D), jnp.float32).at[:, 0].set(1.0)
    zrows = jnp.zeros((RPT, D), jnp.float32)
    xp = jnp.pad(x, ((0, P - N), (0, 0)))
    batch_p = jnp.pad(batch.astype(jnp.int32), (0, P - N),
                      constant_values=G).reshape(1, P)

    _agg_kernel = _sc_kernels()
    degs = _agg_kernel(ones_h, src3, dst3, zrows)
    h0p, dinv = _tc1(xp, W0, degs)
    a0 = _agg_kernel(h0p, src3, dst3, zrows)
    h1p = _tc2(a0, h0p, dinv, b0.reshape(1, D), W1)
    a1 = _agg_kernel(h1p, src3, dst3, zrows)
    h2p = _tc2(a1, h1p, dinv, b1.reshape(1, D), W2)
    a2 = _agg_kernel(h2p, src3, dst3, zrows)
    out = _tc3(a2, h2p, dinv, b2.reshape(1, D), batch_p,
               Wp1, bp1.reshape(1, D), Wp2, bp2.reshape(1, 5))
    return out


# speedup vs baseline: 1.3886x; 1.3886x over previous
import functools

import jax
import jax.numpy as jnp
from jax import lax
from jax.experimental import pallas as pl
from jax.experimental.pallas import tpu as pltpu
from jax.experimental.pallas import tpu_sc as plsc

N = 10000
P = 10240
E = 320000
D = 128
G = 64
NW = 32
CHUNK = 128
NCH0 = 124
NCH1 = 36
TOTCH = 16 * (NCH0 + NCH1)
RPT = P // 16


@functools.lru_cache(maxsize=None)
def _sc_kernels():
    mesh = plsc.VectorSubcoreMesh(core_axis_name="c", subcore_axis_name="s")

    @functools.partial(
        pl.kernel,
        out_type=jax.ShapeDtypeStruct((2, P, D), jnp.float32),
        mesh=mesh,
        scratch_types=[
            pltpu.VMEM((4, CHUNK), jnp.int32),
            pltpu.VMEM((4, CHUNK), jnp.int32),
            pltpu.VMEM((CHUNK, D), jnp.float32),
            pltpu.VMEM((CHUNK, D), jnp.float32),
            pltpu.VMEM_SHARED((P, D), jnp.float32),
            pltpu.SemaphoreType.DMA,
            pltpu.SemaphoreType.DMA,
            pltpu.SemaphoreType.DMA,
            pltpu.SemaphoreType.DMA,
        ],
    )
    def _agg_kernel(h_hbm, src_hbm, dst_hbm, zeros_hbm, out_hbm,
                    sring, dring, rows_a, rows_b, acc,
                    gsem_a, gsem_b, isem_s, isem_d):
        cid = lax.axis_index("c")
        sid = lax.axis_index("s")
        base = jnp.where(cid == 0, sid * NCH0, 16 * NCH0 + sid * NCH1)
        npair = jnp.where(cid == 0, NCH0 // 2, NCH1 // 2)
        pltpu.sync_copy(zeros_hbm, acc.at[pl.ds(sid * RPT, RPT)])
        pltpu.sync_copy(src_hbm.at[pl.ds(base, 2)], sring.at[pl.ds(0, 2)])
        pltpu.sync_copy(dst_hbm.at[pl.ds(base, 2)], dring.at[pl.ds(0, 2)])
        plsc.subcore_barrier()

        @pl.when(npair > 0)
        def _():
            pltpu.async_copy(h_hbm.at[sring.at[0]], rows_a, gsem_a)

        @pl.loop(0, npair)
        def _(k):
            j = base + k * 2
            p = (k % 2) * 2
            q = 2 - p
            more = k + 1 < npair

            @pl.when(more)
            def _():
                pltpu.async_copy(src_hbm.at[pl.ds(j + 2, 2)],
                                 sring.at[pl.ds(q, 2)], isem_s)
                pltpu.async_copy(dst_hbm.at[pl.ds(j + 2, 2)],
                                 dring.at[pl.ds(q, 2)], isem_d)

            pltpu.async_copy(h_hbm.at[sring.at[p + 1]], rows_b, gsem_b)
            pltpu.make_async_copy(h_hbm.at[sring.at[p]], rows_a,
                                  gsem_a).wait()
            pltpu.sync_copy(rows_a, acc.at[dring.at[p]], add=True)

            @pl.when(more)
            def _():
                pltpu.make_async_copy(src_hbm.at[pl.ds(j + 2, 2)],
                                      sring.at[pl.ds(q, 2)], isem_s).wait()
                pltpu.async_copy(h_hbm.at[sring.at[q]], rows_a, gsem_a)

            pltpu.make_async_copy(h_hbm.at[sring.at[p + 1]], rows_b,
                                  gsem_b).wait()
            pltpu.sync_copy(rows_b, acc.at[dring.at[p + 1]], add=True)

            @pl.when(more)
            def _():
                pltpu.make_async_copy(dst_hbm.at[pl.ds(j + 2, 2)],
                                      dring.at[pl.ds(q, 2)], isem_d).wait()

        plsc.subcore_barrier()
        pltpu.sync_copy(acc.at[pl.ds(sid * RPT, RPT)],
                        out_hbm.at[cid, pl.ds(sid * RPT, RPT)])

    @functools.partial(
        pl.kernel,
        out_type=jax.ShapeDtypeStruct((2, P, D), jnp.float32),
        mesh=mesh,
        scratch_types=[
            pltpu.VMEM((4, CHUNK), jnp.int32),
            pltpu.VMEM((CHUNK, D), jnp.float32),
            pltpu.VMEM_SHARED((P, D), jnp.float32),
            pltpu.SemaphoreType.DMA,
        ],
    )
    def _deg_kernel(dst_hbm, erow_hbm, zeros_hbm, out_hbm,
                    dring, rows_c, acc, isem):
        cid = lax.axis_index("c")
        sid = lax.axis_index("s")
        base = jnp.where(cid == 0, sid * NCH0, 16 * NCH0 + sid * NCH1)
        nchk = jnp.where(cid == 0, NCH0, NCH1)
        pltpu.sync_copy(zeros_hbm, acc.at[pl.ds(sid * RPT, RPT)])
        pltpu.sync_copy(erow_hbm, rows_c)
        pltpu.sync_copy(dst_hbm.at[pl.ds(base, 1)], dring.at[pl.ds(0, 1)])
        plsc.subcore_barrier()

        @pl.loop(0, nchk)
        def _(k):
            p = k % 2
            q = 1 - p
            more = k + 1 < nchk

            @pl.when(more)
            def _():
                pltpu.async_copy(dst_hbm.at[pl.ds(base + k + 1, 1)],
                                 dring.at[pl.ds(q, 1)], isem)

            pltpu.sync_copy(rows_c, acc.at[dring.at[p]], add=True)

            @pl.when(more)
            def _():
                pltpu.make_async_copy(dst_hbm.at[pl.ds(base + k + 1, 1)],
                                      dring.at[pl.ds(q, 1)], isem).wait()

        plsc.subcore_barrier()
        pltpu.sync_copy(acc.at[pl.ds(sid * RPT, RPT)],
                        out_hbm.at[cid, pl.ds(sid * RPT, RPT)])

    return _agg_kernel, _deg_kernel


def _tc1_body(x_ref, w_ref, degs_ref, h_ref, dinv_ref):
    deg = degs_ref[0, :, 0:1] + degs_ref[1, :, 0:1] + 1.0
    rows = lax.broadcasted_iota(jnp.int32, (P, 1), 0)
    dinv = jnp.where(rows < N, lax.rsqrt(deg), 0.0)
    h = jnp.dot(x_ref[...], w_ref[...], preferred_element_type=jnp.float32)
    h_ref[...] = h * dinv
    dinv_ref[...] = dinv


def _tc2_body(agg_ref, hp_ref, dinv_ref, b_ref, w_ref, out_ref):
    dinv = dinv_ref[...]
    s = (agg_ref[0] + agg_ref[1] + hp_ref[...]) * dinv + b_ref[...]
    h = jnp.maximum(s, 0.0)
    out_ref[...] = jnp.dot(h, w_ref[...],
                           preferred_element_type=jnp.float32) * dinv


def _tc3_body(agg_ref, hp_ref, dinv_ref, b_ref, batch_ref,
              wp1_ref, bp1_ref, wp2_ref, bp2_ref, out_ref):
    dinv = dinv_ref[...]
    h3 = jnp.maximum(
        (agg_ref[0] + agg_ref[1] + hp_ref[...]) * dinv + b_ref[...], 0.0)
    gids = lax.broadcasted_iota(jnp.int32, (G, P), 0)
    pm = (batch_ref[...] == gids).astype(jnp.float32)
    counts = jnp.sum(pm, axis=1, keepdims=True)
    summed = jnp.dot(pm, h3, preferred_element_type=jnp.float32)
    pooled = summed / jnp.maximum(counts, 1.0)
    t = jnp.maximum(
        jnp.dot(pooled, wp1_ref[...], preferred_element_type=jnp.float32)
        + bp1_ref[...], 0.0)
    out_ref[...] = jnp.dot(t, wp2_ref[...],
                           preferred_element_type=jnp.float32) + bp2_ref[...]


def _tc1(xp, W0, degs):
    return pl.pallas_call(
        _tc1_body,
        out_shape=[jax.ShapeDtypeStruct((P, D), jnp.float32),
                   jax.ShapeDtypeStruct((P, 1), jnp.float32)],
    )(xp, W0, degs)


def _tc2(agg, hp, dinv, b, Wn):
    return pl.pallas_call(
        _tc2_body,
        out_shape=jax.ShapeDtypeStruct((P, D), jnp.float32),
    )(agg, hp, dinv, b, Wn)


def _tc3(agg, hp, dinv, b, batch_p, Wp1, bp1, Wp2, bp2):
    return pl.pallas_call(
        _tc3_body,
        out_shape=jax.ShapeDtypeStruct((G, 5), jnp.float32),
    )(agg, hp, dinv, b, batch_p, Wp1, bp1, Wp2, bp2)


def kernel(x, edge_index, batch, W0, b0, W1, b1, W2, b2, Wp1, bp1, Wp2, bp2):
    ei = edge_index.astype(jnp.int32)
    npad = TOTCH * CHUNK - E
    src = jnp.concatenate([ei[0], jnp.zeros((npad,), jnp.int32)])
    dst = jnp.concatenate([ei[1], jnp.full((npad,), P - 1, jnp.int32)])
    src3 = src.reshape(TOTCH, CHUNK)
    dst3 = dst.reshape(TOTCH, CHUNK)

    erow = jnp.zeros((CHUNK, D), jnp.float32).at[:, 0].set(1.0)
    zrows = jnp.zeros((RPT, D), jnp.float32)
    xp = jnp.pad(x, ((0, P - N), (0, 0)))
    batch_p = jnp.pad(batch.astype(jnp.int32), (0, P - N),
                      constant_values=G).reshape(1, P)

    _agg_kernel, _deg_kernel = _sc_kernels()
    degs = _deg_kernel(dst3, erow, zrows)
    h0p, dinv = _tc1(xp, W0, degs)
    a0 = _agg_kernel(h0p, src3, dst3, zrows)
    h1p = _tc2(a0, h0p, dinv, b0.reshape(1, D), W1)
    a1 = _agg_kernel(h1p, src3, dst3, zrows)
    h2p = _tc2(a1, h1p, dinv, b1.reshape(1, D), W2)
    a2 = _agg_kernel(h2p, src3, dst3, zrows)
    out = _tc3(a2, h2p, dinv, b2.reshape(1, D), batch_p,
               Wp1, bp1.reshape(1, D), Wp2, bp2.reshape(1, 5))
    return out


# speedup vs baseline: 1.4483x; 1.0430x over previous
import functools

import jax
import jax.numpy as jnp
from jax import lax
from jax.experimental import pallas as pl
from jax.experimental.pallas import tpu as pltpu
from jax.experimental.pallas import tpu_sc as plsc

N = 10000
P = 10240
E = 320000
D = 128
G = 64
NW = 32
CHUNK = 128
NCH0 = 150
NCH1 = 10
TOTCH = 16 * (NCH0 + NCH1)
RPT = P // 16


@functools.lru_cache(maxsize=None)
def _sc_kernels():
    mesh = plsc.VectorSubcoreMesh(core_axis_name="c", subcore_axis_name="s")

    @functools.partial(
        pl.kernel,
        out_type=jax.ShapeDtypeStruct((2, P, D), jnp.float32),
        mesh=mesh,
        scratch_types=[
            pltpu.VMEM((4, CHUNK), jnp.int32),
            pltpu.VMEM((4, CHUNK), jnp.int32),
            pltpu.VMEM((CHUNK, D), jnp.float32),
            pltpu.VMEM((CHUNK, D), jnp.float32),
            pltpu.VMEM_SHARED((P, D), jnp.float32),
            pltpu.SemaphoreType.DMA,
            pltpu.SemaphoreType.DMA,
            pltpu.SemaphoreType.DMA,
            pltpu.SemaphoreType.DMA,
        ],
    )
    def _agg_kernel(h_hbm, src_hbm, dst_hbm, zeros_hbm, out_hbm,
                    sring, dring, rows_a, rows_b, acc,
                    gsem_a, gsem_b, isem_s, isem_d):
        cid = lax.axis_index("c")
        sid = lax.axis_index("s")
        base = jnp.where(cid == 0, sid * NCH0, 16 * NCH0 + sid * NCH1)
        npair = jnp.where(cid == 0, NCH0 // 2, NCH1 // 2)
        pltpu.sync_copy(zeros_hbm, acc.at[pl.ds(sid * RPT, RPT)])
        pltpu.sync_copy(src_hbm.at[pl.ds(base, 2)], sring.at[pl.ds(0, 2)])
        pltpu.sync_copy(dst_hbm.at[pl.ds(base, 2)], dring.at[pl.ds(0, 2)])
        plsc.subcore_barrier()

        @pl.when(npair > 0)
        def _():
            pltpu.async_copy(h_hbm.at[sring.at[0]], rows_a, gsem_a)

        @pl.loop(0, npair)
        def _(k):
            j = base + k * 2
            p = (k % 2) * 2
            q = 2 - p
            more = k + 1 < npair

            @pl.when(more)
            def _():
                pltpu.async_copy(src_hbm.at[pl.ds(j + 2, 2)],
                                 sring.at[pl.ds(q, 2)], isem_s)
                pltpu.async_copy(dst_hbm.at[pl.ds(j + 2, 2)],
                                 dring.at[pl.ds(q, 2)], isem_d)

            pltpu.async_copy(h_hbm.at[sring.at[p + 1]], rows_b, gsem_b)
            pltpu.make_async_copy(h_hbm.at[sring.at[p]], rows_a,
                                  gsem_a).wait()
            pltpu.sync_copy(rows_a, acc.at[dring.at[p]], add=True)

            @pl.when(more)
            def _():
                pltpu.make_async_copy(src_hbm.at[pl.ds(j + 2, 2)],
                                      sring.at[pl.ds(q, 2)], isem_s).wait()
                pltpu.async_copy(h_hbm.at[sring.at[q]], rows_a, gsem_a)

            pltpu.make_async_copy(h_hbm.at[sring.at[p + 1]], rows_b,
                                  gsem_b).wait()
            pltpu.sync_copy(rows_b, acc.at[dring.at[p + 1]], add=True)

            @pl.when(more)
            def _():
                pltpu.make_async_copy(dst_hbm.at[pl.ds(j + 2, 2)],
                                      dring.at[pl.ds(q, 2)], isem_d).wait()

        plsc.subcore_barrier()
        pltpu.sync_copy(acc.at[pl.ds(sid * RPT, RPT)],
                        out_hbm.at[cid, pl.ds(sid * RPT, RPT)])

    @functools.partial(
        pl.kernel,
        out_type=jax.ShapeDtypeStruct((2, P, D), jnp.float32),
        mesh=mesh,
        scratch_types=[
            pltpu.VMEM((4, CHUNK), jnp.int32),
            pltpu.VMEM((CHUNK, D), jnp.float32),
            pltpu.VMEM_SHARED((P, D), jnp.float32),
            pltpu.SemaphoreType.DMA,
        ],
    )
    def _deg_kernel(dst_hbm, erow_hbm, zeros_hbm, out_hbm,
                    dring, rows_c, acc, isem):
        cid = lax.axis_index("c")
        sid = lax.axis_index("s")
        base = jnp.where(cid == 0, sid * NCH0, 16 * NCH0 + sid * NCH1)
        nchk = jnp.where(cid == 0, NCH0, NCH1)
        pltpu.sync_copy(zeros_hbm, acc.at[pl.ds(sid * RPT, RPT)])
        pltpu.sync_copy(erow_hbm, rows_c)
        pltpu.sync_copy(dst_hbm.at[pl.ds(base, 1)], dring.at[pl.ds(0, 1)])
        plsc.subcore_barrier()

        @pl.loop(0, nchk)
        def _(k):
            p = k % 2
            q = 1 - p
            more = k + 1 < nchk

            @pl.when(more)
            def _():
                pltpu.async_copy(dst_hbm.at[pl.ds(base + k + 1, 1)],
                                 dring.at[pl.ds(q, 1)], isem)

            pltpu.sync_copy(rows_c, acc.at[dring.at[p]], add=True)

            @pl.when(more)
            def _():
                pltpu.make_async_copy(dst_hbm.at[pl.ds(base + k + 1, 1)],
                                      dring.at[pl.ds(q, 1)], isem).wait()

        plsc.subcore_barrier()
        pltpu.sync_copy(acc.at[pl.ds(sid * RPT, RPT)],
                        out_hbm.at[cid, pl.ds(sid * RPT, RPT)])

    return _agg_kernel, _deg_kernel


def _tc1_body(x_ref, w_ref, degs_ref, h_ref, dinv_ref):
    deg = degs_ref[0, :, 0:1] + degs_ref[1, :, 0:1] + 1.0
    rows = lax.broadcasted_iota(jnp.int32, (P, 1), 0)
    dinv = jnp.where(rows < N, lax.rsqrt(deg), 0.0)
    h = jnp.dot(x_ref[...], w_ref[...], preferred_element_type=jnp.float32)
    h_ref[...] = h * dinv
    dinv_ref[...] = dinv


def _tc2_body(agg_ref, hp_ref, dinv_ref, b_ref, w_ref, out_ref):
    dinv = dinv_ref[...]
    s = (agg_ref[0] + agg_ref[1] + hp_ref[...]) * dinv + b_ref[...]
    h = jnp.maximum(s, 0.0)
    out_ref[...] = jnp.dot(h, w_ref[...],
                           preferred_element_type=jnp.float32) * dinv


def _tc3_body(agg_ref, hp_ref, dinv_ref, b_ref, batch_ref,
              wp1_ref, bp1_ref, wp2_ref, bp2_ref, out_ref):
    dinv = dinv_ref[...]
    h3 = jnp.maximum(
        (agg_ref[0] + agg_ref[1] + hp_ref[...]) * dinv + b_ref[...], 0.0)
    gids = lax.broadcasted_iota(jnp.int32, (G, P), 0)
    pm = (batch_ref[...] == gids).astype(jnp.float32)
    counts = jnp.sum(pm, axis=1, keepdims=True)
    summed = jnp.dot(pm, h3, preferred_element_type=jnp.float32)
    pooled = summed / jnp.maximum(counts, 1.0)
    t = jnp.maximum(
        jnp.dot(pooled, wp1_ref[...], preferred_element_type=jnp.float32)
        + bp1_ref[...], 0.0)
    out_ref[...] = jnp.dot(t, wp2_ref[...],
                           preferred_element_type=jnp.float32) + bp2_ref[...]


def _tc1(xp, W0, degs):
    return pl.pallas_call(
        _tc1_body,
        out_shape=[jax.ShapeDtypeStruct((P, D), jnp.float32),
                   jax.ShapeDtypeStruct((P, 1), jnp.float32)],
    )(xp, W0, degs)


def _tc2(agg, hp, dinv, b, Wn):
    return pl.pallas_call(
        _tc2_body,
        out_shape=jax.ShapeDtypeStruct((P, D), jnp.float32),
    )(agg, hp, dinv, b, Wn)


def _tc3(agg, hp, dinv, b, batch_p, Wp1, bp1, Wp2, bp2):
    return pl.pallas_call(
        _tc3_body,
        out_shape=jax.ShapeDtypeStruct((G, 5), jnp.float32),
    )(agg, hp, dinv, b, batch_p, Wp1, bp1, Wp2, bp2)


def kernel(x, edge_index, batch, W0, b0, W1, b1, W2, b2, Wp1, bp1, Wp2, bp2):
    ei = edge_index.astype(jnp.int32)
    npad = TOTCH * CHUNK - E
    src = jnp.concatenate([ei[0], jnp.zeros((npad,), jnp.int32)])
    dst = jnp.concatenate([ei[1], jnp.full((npad,), P - 1, jnp.int32)])
    src3 = src.reshape(TOTCH, CHUNK)
    dst3 = dst.reshape(TOTCH, CHUNK)

    erow = jnp.zeros((CHUNK, D), jnp.float32).at[:, 0].set(1.0)
    zrows = jnp.zeros((RPT, D), jnp.float32)
    xp = jnp.pad(x, ((0, P - N), (0, 0)))
    batch_p = jnp.pad(batch.astype(jnp.int32), (0, P - N),
                      constant_values=G).reshape(1, P)

    _agg_kernel, _deg_kernel = _sc_kernels()
    degs = _deg_kernel(dst3, erow, zrows)
    h0p, dinv = _tc1(xp, W0, degs)
    a0 = _agg_kernel(h0p, src3, dst3, zrows)
    h1p = _tc2(a0, h0p, dinv, b0.reshape(1, D), W1)
    a1 = _agg_kernel(h1p, src3, dst3, zrows)
    h2p = _tc2(a1, h1p, dinv, b1.reshape(1, D), W2)
    a2 = _agg_kernel(h2p, src3, dst3, zrows)
    out = _tc3(a2, h2p, dinv, b2.reshape(1, D), batch_p,
               Wp1, bp1.reshape(1, D), Wp2, bp2.reshape(1, 5))
    return out
